# Initial kernel scaffold; baseline (speedup 1.0000x reference)
import jax
import jax.numpy as jnp
from jax.experimental import pallas as pl


def kernel(x, embeddings):
    raise NotImplementedError("write your pallas kernel here")


# speedup vs baseline: 1.4939x; 1.4939x over previous
import functools

import jax
import jax.numpy as jnp
from jax import lax
from jax.experimental import pallas as pl
from jax.experimental.pallas import tpu as pltpu
from jax.experimental.pallas import tpu_sc as plsc

_N_TOKENS = 16384
_NUM_EMB = 8192
_DIM = 32
_TB = 256


def _argmin_body(x_ref, e_ref, idx_ref):
    xb = x_ref[...]
    e = e_ref[...]
    e2 = jnp.sum(e * e, axis=0, keepdims=True)
    d = e2 - 2.0 * jnp.dot(xb, e, preferred_element_type=jnp.float32)
    m = jnp.min(d, axis=1, keepdims=True)
    ii = lax.broadcasted_iota(jnp.int32, d.shape, 1)
    idx = jnp.min(jnp.where(d == m, ii, _NUM_EMB), axis=1)
    idx_ref[0, 0, :] = idx.astype(jnp.int32)


def _tc_argmin(x, embeddings):
    nb = _N_TOKENS // _TB
    out = pl.pallas_call(
        _argmin_body,
        grid=(nb,),
        in_specs=[
            pl.BlockSpec((_TB, _DIM), lambda i: (i, 0)),
            pl.BlockSpec((_DIM, _NUM_EMB), lambda i: (0, 0)),
        ],
        out_specs=pl.BlockSpec((1, 1, _TB), lambda i: (i, 0, 0)),
        out_shape=jax.ShapeDtypeStruct((nb, 1, _TB), jnp.int32),
    )(x, embeddings)
    return out.reshape(_N_TOKENS)


def _sc_gather(table, idx):
    info = plsc.get_sparse_core_info()
    nc, ns = info.num_cores, info.num_subcores
    nw = nc * ns
    bpw = _N_TOKENS // nw
    mesh = plsc.VectorSubcoreMesh(core_axis_name="c", subcore_axis_name="s")

    @functools.partial(
        pl.kernel,
        mesh=mesh,
        compiler_params=pltpu.CompilerParams(use_tc_tiling_on_sc=False),
        out_type=jax.ShapeDtypeStruct((_N_TOKENS, _DIM), jnp.float32),
        scratch_types=[
            pltpu.VMEM((bpw,), jnp.int32),
            pltpu.VMEM((bpw, _DIM), jnp.float32),
            pltpu.SemaphoreType.DMA,
        ],
    )
    def gather_kernel(table_hbm, idx_hbm, out_hbm, idx_v, rows_v, sem):
        wid = lax.axis_index("s") * nc + lax.axis_index("c")
        base = wid * bpw
        pltpu.sync_copy(idx_hbm.at[pl.ds(base, bpw)], idx_v)
        pltpu.async_copy(table_hbm.at[idx_v], rows_v, sem).wait()
        pltpu.sync_copy(rows_v, out_hbm.at[pl.ds(base, bpw)])

    return gather_kernel(table, idx)


def kernel(x, embeddings):
    idx = _tc_argmin(x, embeddings)
    table = embeddings.T
    return _sc_gather(table, idx)


# speedup vs baseline: 2.3049x; 1.5429x over previous
import functools

import jax
import jax.numpy as jnp
from jax import lax
from jax.experimental import pallas as pl
from jax.experimental.pallas import tpu as pltpu
from jax.experimental.pallas import tpu_sc as plsc

_N_TOKENS = 16384
_NUM_EMB = 8192
_DIM = 32
_TB = 256


_LANES = 128


def _argmin_body(x_ref, e_ref, idx_ref, eaug_ref):
    @pl.when(pl.program_id(0) == 0)
    def _init():
        e = e_ref[...]
        eaug_ref[:_DIM, :] = e * -2.0
        eaug_ref[_DIM:, :] = jnp.sum(e * e, axis=0, keepdims=True)

    xa = jnp.concatenate(
        [x_ref[...], jnp.ones((_TB, 1), jnp.float32)], axis=1
    )
    d = jnp.dot(xa, eaug_ref[...], preferred_element_type=jnp.float32)

    run_min = d[:, :_LANES]
    run_cid = jnp.zeros((_TB, _LANES), jnp.float32)
    for c in range(1, _NUM_EMB // _LANES):
        dc = d[:, c * _LANES:(c + 1) * _LANES]
        pred = dc < run_min
        run_min = jnp.where(pred, dc, run_min)
        run_cid = jnp.where(pred, jnp.float32(c), run_cid)

    m = jnp.min(run_min, axis=1, keepdims=True)
    lane = lax.broadcasted_iota(
        jnp.int32, (_TB, _LANES), 1
    ).astype(jnp.float32)
    cand = jnp.where(
        run_min == m, run_cid * _LANES + lane, jnp.float32(_NUM_EMB)
    )
    idx_ref[0, 0, :] = jnp.min(cand, axis=1).astype(jnp.int32)


def _tc_argmin(x, embeddings):
    nb = _N_TOKENS // _TB
    out = pl.pallas_call(
        _argmin_body,
        grid=(nb,),
        in_specs=[
            pl.BlockSpec((_TB, _DIM), lambda i: (i, 0)),
            pl.BlockSpec((_DIM, _NUM_EMB), lambda i: (0, 0)),
        ],
        out_specs=pl.BlockSpec((1, 1, _TB), lambda i: (i, 0, 0)),
        out_shape=jax.ShapeDtypeStruct((nb, 1, _TB), jnp.int32),
        scratch_shapes=[pltpu.VMEM((_DIM + 1, _NUM_EMB), jnp.float32)],
    )(x, embeddings)
    return out.reshape(_N_TOKENS)


def _sc_gather(table, idx):
    info = plsc.get_sparse_core_info()
    nc, ns = info.num_cores, info.num_subcores
    nw = nc * ns
    bpw = _N_TOKENS // nw
    mesh = plsc.VectorSubcoreMesh(core_axis_name="c", subcore_axis_name="s")

    @functools.partial(
        pl.kernel,
        mesh=mesh,
        compiler_params=pltpu.CompilerParams(use_tc_tiling_on_sc=False),
        out_type=jax.ShapeDtypeStruct((_N_TOKENS, _DIM), jnp.float32),
        scratch_types=[
            pltpu.VMEM((bpw,), jnp.int32),
            pltpu.VMEM((bpw, _DIM), jnp.float32),
            pltpu.SemaphoreType.DMA,
        ],
    )
    def gather_kernel(table_hbm, idx_hbm, out_hbm, idx_v, rows_v, sem):
        wid = lax.axis_index("s") * nc + lax.axis_index("c")
        base = wid * bpw
        pltpu.sync_copy(idx_hbm.at[pl.ds(base, bpw)], idx_v)
        pltpu.async_copy(table_hbm.at[idx_v], rows_v, sem).wait()
        pltpu.sync_copy(rows_v, out_hbm.at[pl.ds(base, bpw)])

    return gather_kernel(table, idx)


def kernel(x, embeddings):
    idx = _tc_argmin(x, embeddings)
    table = embeddings.T
    return _sc_gather(table, idx)
